# Initial kernel scaffold; baseline (speedup 1.0000x reference)
import jax
import jax.numpy as jnp
from jax.experimental import pallas as pl


def kernel(x, edge_index, W_fc, W_attn):
    raise NotImplementedError("write your pallas kernel here")


# speedup vs baseline: 8.4318x; 8.4318x over previous
import functools

import jax
import jax.numpy as jnp
from jax import lax
from jax.experimental import pallas as pl
from jax.experimental.pallas import tpu as pltpu
from jax.experimental.pallas import tpu_sc as plsc

N = 10000
D = 128
E = 320000

NC = 2
NS = 16
NW = NC * NS
EPW = E // NW
CH = 80
NCHUNK = EPW // CH
RPS = 624
TAIL = N - NS * RPS


def _mm_body(x_ref, wt_ref, z_ref):
    z_ref[...] = jnp.dot(x_ref[...], wt_ref[...],
                         preferred_element_type=jnp.float32)


def _matmul(x, w_t):
    return pl.pallas_call(
        _mm_body,
        grid=(10,),
        in_specs=[
            pl.BlockSpec((N // 10, D), lambda i: (i, 0)),
            pl.BlockSpec((D, D), lambda i: (0, 0)),
        ],
        out_specs=pl.BlockSpec((N // 10, D), lambda i: (i, 0)),
        out_shape=jax.ShapeDtypeStruct((N, D), jnp.float32),
    )(x, w_t)


_MESH = plsc.VectorSubcoreMesh(core_axis_name="c", subcore_axis_name="s")


@functools.partial(
    pl.kernel,
    out_type=jax.ShapeDtypeStruct((NC, N, D), jnp.float32),
    mesh=_MESH,
    scratch_types=[
        pltpu.VMEM((CH,), jnp.int32),
        pltpu.VMEM((CH,), jnp.int32),
        pltpu.VMEM((CH, D), jnp.float32),
        pltpu.VMEM_SHARED((N, D), jnp.float32),
        pltpu.SemaphoreType.DMA,
    ],
)
def _sc_scatter(z_hbm, row_hbm, col_hbm, zeros_hbm, out_hbm,
                col_v, row_v, rows_v, acc_sh, sem):
    c = lax.axis_index("c")
    s = lax.axis_index("s")
    wid = s * NC + c

    stripe = pl.ds(pl.multiple_of(s * RPS, 8), RPS)
    tail = pl.ds(NS * RPS, TAIL)
    pltpu.sync_copy(zeros_hbm.at[stripe], acc_sh.at[stripe])

    @pl.when(s == 0)
    def _zero_tail():
        pltpu.sync_copy(zeros_hbm.at[tail], acc_sh.at[tail])

    plsc.subcore_barrier()

    base = wid * EPW

    def body(i, _):
        off = pl.multiple_of(base + i * CH, 8)
        pltpu.sync_copy(col_hbm.at[pl.ds(off, CH)], col_v)
        pltpu.sync_copy(row_hbm.at[pl.ds(off, CH)], row_v)
        pltpu.async_copy(z_hbm.at[col_v], rows_v, sem).wait()
        pltpu.sync_copy(rows_v, acc_sh.at[row_v], add=True)
        return 0

    lax.fori_loop(0, NCHUNK, body, 0)
    plsc.subcore_barrier()

    pltpu.sync_copy(acc_sh.at[stripe], out_hbm.at[c].at[stripe])

    @pl.when(s == 0)
    def _copy_tail():
        pltpu.sync_copy(acc_sh.at[tail], out_hbm.at[c].at[tail])


def _add_body(p_ref, o_ref):
    o_ref[...] = p_ref[0] + p_ref[1]


def _combine(partials):
    return pl.pallas_call(
        _add_body,
        grid=(10,),
        in_specs=[pl.BlockSpec((NC, N // 10, D), lambda i: (0, i, 0))],
        out_specs=pl.BlockSpec((N // 10, D), lambda i: (i, 0)),
        out_shape=jax.ShapeDtypeStruct((N, D), jnp.float32),
    )(partials)


def kernel(x, edge_index, W_fc, W_attn):
    z = _matmul(x, W_fc.T)
    row = edge_index[0]
    col = edge_index[1]
    zeros = jnp.zeros((N, D), dtype=jnp.float32)
    partials = _sc_scatter(z, row, col, zeros)
    return _combine(partials)


# speedup vs baseline: 16.5266x; 1.9600x over previous
import functools

import jax
import jax.numpy as jnp
from jax import lax
from jax.experimental import pallas as pl
from jax.experimental.pallas import tpu as pltpu
from jax.experimental.pallas import tpu_sc as plsc

N = 10000
D = 128
E = 320000

NC = 2
NS = 16
NW = NC * NS
EPW = E // NW
CH = 80
NCHUNK = EPW // CH
G = 25
NG = NCHUNK // G
RPS = 624
TAIL = N - NS * RPS


def _mm_body(x_ref, wt_ref, z_ref):
    z_ref[...] = jnp.dot(x_ref[...], wt_ref[...],
                         preferred_element_type=jnp.float32)


def _matmul(x, w_t):
    return pl.pallas_call(
        _mm_body,
        grid=(10,),
        in_specs=[
            pl.BlockSpec((N // 10, D), lambda i: (i, 0)),
            pl.BlockSpec((D, D), lambda i: (0, 0)),
        ],
        out_specs=pl.BlockSpec((N // 10, D), lambda i: (i, 0)),
        out_shape=jax.ShapeDtypeStruct((N, D), jnp.float32),
    )(x, w_t)


_MESH = plsc.VectorSubcoreMesh(core_axis_name="c", subcore_axis_name="s")


@functools.partial(
    pl.kernel,
    out_type=jax.ShapeDtypeStruct((NC, N, D), jnp.float32),
    mesh=_MESH,
    scratch_types=[
        pltpu.VMEM((G, CH), jnp.int32),
        pltpu.VMEM((G, CH), jnp.int32),
        pltpu.VMEM((CH, D), jnp.float32),
        pltpu.VMEM((CH, D), jnp.float32),
        pltpu.VMEM_SHARED((N, D), jnp.float32),
        pltpu.SemaphoreType.DMA,
        pltpu.SemaphoreType.DMA,
        pltpu.SemaphoreType.DMA,
        pltpu.SemaphoreType.DMA,
    ],
)
def _sc_scatter(z_hbm, row_hbm, col_hbm, zeros_hbm, out_hbm,
                col_v, row_v, buf0, buf1, acc_sh, gs0, gs1, ss0, ss1):
    c = lax.axis_index("c")
    s = lax.axis_index("s")
    wid = s * NC + c

    stripe = pl.ds(pl.multiple_of(s * RPS, 8), RPS)
    tail = pl.ds(NS * RPS, TAIL)
    pltpu.sync_copy(zeros_hbm.at[stripe], acc_sh.at[stripe])

    @pl.when(s == 0)
    def _zero_tail():
        pltpu.sync_copy(zeros_hbm.at[tail], acc_sh.at[tail])

    plsc.subcore_barrier()

    for g in range(NG):
        pltpu.sync_copy(col_hbm.at[wid].at[g], col_v)
        pltpu.sync_copy(row_hbm.at[wid].at[g], row_v)
        pltpu.async_copy(z_hbm.at[col_v.at[0]], buf0, gs0)

        def _pipe(jj, _):
            @pl.when(jj > 0)
            def _w0():
                pltpu.make_async_copy(buf1, acc_sh.at[row_v.at[jj - 1]],
                                      ss1).wait()

            pltpu.async_copy(z_hbm.at[col_v.at[jj + 1]], buf1, gs1)
            pltpu.make_async_copy(z_hbm.at[col_v.at[jj]], buf0, gs0).wait()
            pltpu.async_copy(buf0, acc_sh.at[row_v.at[jj]], ss0, add=True)
            pltpu.make_async_copy(buf0, acc_sh.at[row_v.at[jj]], ss0).wait()
            pltpu.async_copy(z_hbm.at[col_v.at[jj + 2]], buf0, gs0)
            pltpu.make_async_copy(z_hbm.at[col_v.at[jj + 1]], buf1,
                                  gs1).wait()
            pltpu.async_copy(buf1, acc_sh.at[row_v.at[jj + 1]], ss1,
                             add=True)
            return 0

        lax.fori_loop(0, (G - 1) // 2, lambda i, cy: _pipe(2 * i, cy), 0)

        last = G - 1
        pltpu.make_async_copy(buf1, acc_sh.at[row_v.at[last - 1]],
                              ss1).wait()
        pltpu.make_async_copy(z_hbm.at[col_v.at[last]], buf0, gs0).wait()
        pltpu.async_copy(buf0, acc_sh.at[row_v.at[last]], ss0, add=True)
        pltpu.make_async_copy(buf0, acc_sh.at[row_v.at[last]], ss0).wait()

    plsc.subcore_barrier()

    pltpu.sync_copy(acc_sh.at[stripe], out_hbm.at[c].at[stripe])

    @pl.when(s == 0)
    def _copy_tail():
        pltpu.sync_copy(acc_sh.at[tail], out_hbm.at[c].at[tail])


def _add_body(p_ref, o_ref):
    o_ref[...] = p_ref[0] + p_ref[1]


def _combine(partials):
    return pl.pallas_call(
        _add_body,
        grid=(10,),
        in_specs=[pl.BlockSpec((NC, N // 10, D), lambda i: (0, i, 0))],
        out_specs=pl.BlockSpec((N // 10, D), lambda i: (i, 0)),
        out_shape=jax.ShapeDtypeStruct((N, D), jnp.float32),
    )(partials)


def kernel(x, edge_index, W_fc, W_attn):
    z = _matmul(x, W_fc.T)
    row = edge_index[0].reshape(NW, NG, G, CH)
    col = edge_index[1].reshape(NW, NG, G, CH)
    zeros = jnp.zeros((N, D), dtype=jnp.float32)
    partials = _sc_scatter(z, row, col, zeros)
    return _combine(partials)
